# Initial kernel scaffold; baseline (speedup 1.0000x reference)
import jax
import jax.numpy as jnp
from jax.experimental import pallas as pl


def kernel(x, edge_index, batch, edge_seq, edge_seq_lengths, cheb_W0, cheb_W1, cheb_b, gat_W, gat_att_src, gat_att_dst, gat_b, W_ih, W_hh, b_ih, b_hh, lin_W, lin_b):
    raise NotImplementedError("write your pallas kernel here")


# speedup vs baseline: 31.4670x; 31.4670x over previous
import functools

import jax
import jax.numpy as jnp
from jax import lax
from jax.experimental import pallas as pl
from jax.experimental.pallas import tpu as pltpu
from jax.experimental.pallas import tpu_sc as plsc

N = 10000
NPAD = 10240
E = 320000
D = 128
NG = 16
SEQ = 200
LH = 128

NC = 2
NS = 16
EPT = E // (NC * NS)
CHUNK = 80
NCHUNKS = EPT // CHUNK
NPAIRS = (NCHUNKS - 1) // 2
NODES_PER_TILE = NPAD // NS
BLK = 1024
NBLK = NPAD // BLK

_SC_MESH = plsc.VectorSubcoreMesh(core_axis_name="c", subcore_axis_name="s")


def _edge_slice(ebase, k):
    return pl.ds(ebase + k * CHUNK, CHUNK)


@functools.partial(
    pl.kernel,
    out_type=jax.ShapeDtypeStruct((NC, NPAD), jnp.float32),
    mesh=_SC_MESH,
    scratch_types=[
        pltpu.VMEM_SHARED((NPAD,), jnp.float32),
        pltpu.VMEM((CHUNK,), jnp.int32),
        pltpu.VMEM((CHUNK,), jnp.int32),
        pltpu.VMEM((CHUNK,), jnp.float32),
        pltpu.SemaphoreType.DMA,
    ],
)
def _sc_deg(erow_hbm, zn_hbm, out_hbm, deg_sh, idx0, idx1, ones_v, isem):
    c = lax.axis_index("c")
    s = lax.axis_index("s")
    nbase = s * NODES_PER_TILE
    pltpu.sync_copy(zn_hbm.at[pl.ds(nbase, NODES_PER_TILE)],
                    deg_sh.at[pl.ds(nbase, NODES_PER_TILE)])

    def fill(j, _):
        ones_v[pl.ds(j * 16, 16)] = jnp.ones((16,), jnp.float32)
        return 0

    lax.fori_loop(0, CHUNK // 16, fill, 0)
    plsc.subcore_barrier()
    ebase = c * (E // NC) + s * EPT
    idx = (idx0, idx1)
    pltpu.async_copy(erow_hbm.at[_edge_slice(ebase, 0)], idx0, isem)

    def pair(kk, _):
        for b in (0, 1):
            k = 2 * kk + b
            nb = 1 - b
            pltpu.make_async_copy(
                erow_hbm.at[_edge_slice(ebase, k)], idx[b], isem).wait()
            pltpu.async_copy(
                erow_hbm.at[_edge_slice(ebase, k + 1)], idx[nb], isem)
            pltpu.sync_copy(ones_v, deg_sh.at[idx[b]], add=True)
        return 0

    lax.fori_loop(0, NPAIRS, pair, 0)
    pltpu.make_async_copy(
        erow_hbm.at[_edge_slice(ebase, NCHUNKS - 1)], idx0, isem).wait()
    pltpu.sync_copy(ones_v, deg_sh.at[idx0], add=True)
    plsc.subcore_barrier()
    pltpu.sync_copy(deg_sh.at[pl.ds(nbase, NODES_PER_TILE)],
                    out_hbm.at[c, pl.ds(nbase, NODES_PER_TILE)])


@functools.partial(
    pl.kernel,
    out_type=jax.ShapeDtypeStruct((NC, NPAD, D), jnp.float32),
    mesh=_SC_MESH,
    scratch_types=[
        pltpu.VMEM_SHARED((NPAD, D), jnp.float32),
        pltpu.VMEM((CHUNK,), jnp.int32),
        pltpu.VMEM((CHUNK,), jnp.int32),
        pltpu.VMEM((CHUNK,), jnp.int32),
        pltpu.VMEM((CHUNK,), jnp.int32),
        pltpu.VMEM((CHUNK, D), jnp.float32),
        pltpu.VMEM((CHUNK, D), jnp.float32),
        pltpu.SemaphoreType.DMA,
    ],
)
def _sc_cheb(y_hbm, erow_hbm, ecol_hbm, znd_hbm, out_hbm, acc_sh,
             ridx0, ridx1, cidx0, cidx1, rows0, rows1, gsem):
    c = lax.axis_index("c")
    s = lax.axis_index("s")
    nbase = s * NODES_PER_TILE
    pltpu.sync_copy(znd_hbm.at[pl.ds(nbase, NODES_PER_TILE)],
                    acc_sh.at[pl.ds(nbase, NODES_PER_TILE)])
    plsc.subcore_barrier()
    ebase = c * (E // NC) + s * EPT
    ridx = (ridx0, ridx1)
    cidx = (cidx0, cidx1)
    rows = (rows0, rows1)
    pltpu.sync_copy(erow_hbm.at[_edge_slice(ebase, 0)], ridx0)
    pltpu.sync_copy(ecol_hbm.at[_edge_slice(ebase, 0)], cidx0)
    pltpu.async_copy(y_hbm.at[ridx0], rows0, gsem)

    def pair(kk, _):
        for b in (0, 1):
            k = 2 * kk + b
            nb = 1 - b
            pltpu.sync_copy(erow_hbm.at[_edge_slice(ebase, k + 1)], ridx[nb])
            pltpu.sync_copy(ecol_hbm.at[_edge_slice(ebase, k + 1)], cidx[nb])
            pltpu.make_async_copy(y_hbm.at[ridx[b]], rows[b], gsem).wait()
            pltpu.async_copy(y_hbm.at[ridx[nb]], rows[nb], gsem)
            pltpu.sync_copy(rows[b], acc_sh.at[cidx[b]], add=True)
        return 0

    lax.fori_loop(0, NPAIRS, pair, 0)
    pltpu.make_async_copy(y_hbm.at[ridx0], rows0, gsem).wait()
    pltpu.sync_copy(rows0, acc_sh.at[cidx0], add=True)
    plsc.subcore_barrier()
    pltpu.sync_copy(acc_sh.at[pl.ds(nbase, NODES_PER_TILE)],
                    out_hbm.at[c, pl.ds(nbase, NODES_PER_TILE)])


@functools.partial(
    pl.kernel,
    out_type=(
        jax.ShapeDtypeStruct((NC, NPAD, D), jnp.float32),
        jax.ShapeDtypeStruct((NC, NPAD), jnp.float32),
    ),
    mesh=_SC_MESH,
    scratch_types=[
        pltpu.VMEM_SHARED((NPAD, D), jnp.float32),
        pltpu.VMEM_SHARED((NPAD,), jnp.float32),
        pltpu.VMEM((CHUNK,), jnp.int32),
        pltpu.VMEM((CHUNK,), jnp.int32),
        pltpu.VMEM((CHUNK,), jnp.int32),
        pltpu.VMEM((CHUNK,), jnp.int32),
        pltpu.VMEM((CHUNK, D), jnp.float32),
        pltpu.VMEM((CHUNK, D), jnp.float32),
        pltpu.VMEM((CHUNK,), jnp.float32),
        pltpu.VMEM((CHUNK,), jnp.float32),
        pltpu.VMEM((CHUNK,), jnp.float32),
        pltpu.VMEM((CHUNK,), jnp.float32),
        pltpu.VMEM((CHUNK,), jnp.float32),
        pltpu.VMEM((16,), jnp.float32),
        pltpu.SemaphoreType.DMA,
        pltpu.SemaphoreType.DMA,
        pltpu.SemaphoreType.DMA,
    ],
)
def _sc_gat(xw_hbm, erow_hbm, ecol_hbm, asrc_hbm, adst_hbm, mx_hbm, znd_hbm,
            zn_hbm, num_hbm, den_hbm,
            acc_sh, den_sh, ridx0, ridx1, cidx0, cidx1, rows0, rows1,
            av0, av1, dv0, dv1, u_v, mx_v, asem, dsem, rsem):
    c = lax.axis_index("c")
    s = lax.axis_index("s")
    nbase = s * NODES_PER_TILE
    pltpu.sync_copy(znd_hbm.at[pl.ds(nbase, NODES_PER_TILE)],
                    acc_sh.at[pl.ds(nbase, NODES_PER_TILE)])
    pltpu.sync_copy(zn_hbm.at[pl.ds(nbase, NODES_PER_TILE)],
                    den_sh.at[pl.ds(nbase, NODES_PER_TILE)])
    pltpu.sync_copy(mx_hbm, mx_v)
    plsc.subcore_barrier()
    ebase = c * (E // NC) + s * EPT
    ridx = (ridx0, ridx1)
    cidx = (cidx0, cidx1)
    rows = (rows0, rows1)
    av = (av0, av1)
    dv = (dv0, dv1)
    pltpu.sync_copy(erow_hbm.at[_edge_slice(ebase, 0)], ridx0)
    pltpu.sync_copy(ecol_hbm.at[_edge_slice(ebase, 0)], cidx0)
    pltpu.async_copy(asrc_hbm.at[ridx0], av0, asem)
    pltpu.async_copy(adst_hbm.at[cidx0], dv0, dsem)
    pltpu.async_copy(xw_hbm.at[ridx0], rows0, rsem)

    def process(b, prefetch, k):
        nb = 1 - b
        pltpu.make_async_copy(asrc_hbm.at[ridx[b]], av[b], asem).wait()
        pltpu.make_async_copy(adst_hbm.at[cidx[b]], dv[b], dsem).wait()
        if prefetch:
            pltpu.sync_copy(erow_hbm.at[_edge_slice(ebase, k + 1)], ridx[nb])
            pltpu.sync_copy(ecol_hbm.at[_edge_slice(ebase, k + 1)], cidx[nb])
            pltpu.async_copy(asrc_hbm.at[ridx[nb]], av[nb], asem)
            pltpu.async_copy(adst_hbm.at[cidx[nb]], dv[nb], dsem)
        mx = mx_v[...]

        @plsc.parallel_loop(0, CHUNK // 16, 1, unroll=CHUNK // 16)
        def _(j):
            sl = pl.ds(j * 16, 16)
            a = av[b][sl] + dv[b][sl]
            alpha = jnp.where(a > 0, a, 0.2 * a)
            m = mx + dv[b][sl]
            m = jnp.where(m > 0, m, 0.2 * m)
            u_v[sl] = jnp.exp(alpha - m)

        pltpu.sync_copy(u_v, den_sh.at[cidx[b]], add=True)
        pltpu.make_async_copy(xw_hbm.at[ridx[b]], rows[b], rsem).wait()
        if prefetch:
            pltpu.async_copy(xw_hbm.at[ridx[nb]], rows[nb], rsem)

        @plsc.parallel_loop(0, CHUNK // 16, 1, unroll=CHUNK // 16)
        def _(g):
            uvec = u_v[pl.ds(g * 16, 16)]
            for r in range(16):
                uu = uvec[r]
                for j in range(D // 16):
                    sl = pl.ds(j * 16, 16)
                    rows[b][g * 16 + r, sl] = rows[b][g * 16 + r, sl] * uu

        pltpu.sync_copy(rows[b], acc_sh.at[cidx[b]], add=True)

    def pair(kk, _):
        for b in (0, 1):
            process(b, True, 2 * kk + b)
        return 0

    lax.fori_loop(0, NPAIRS, pair, 0)
    process(0, False, NCHUNKS - 1)
    plsc.subcore_barrier()
    pltpu.sync_copy(acc_sh.at[pl.ds(nbase, NODES_PER_TILE)],
                    num_hbm.at[c, pl.ds(nbase, NODES_PER_TILE)])
    pltpu.sync_copy(den_sh.at[pl.ds(nbase, NODES_PER_TILE)],
                    den_hbm.at[c, pl.ds(nbase, NODES_PER_TILE)])


def _tc_prep_body(deg_ref, x_ref, dinv_ref, y_ref):
    d = deg_ref[0] + deg_ref[1]
    dinv = jnp.where(d > 0, 1.0 / jnp.sqrt(jnp.maximum(d, 1e-12)), 0.0)
    dinv_ref[...] = dinv
    y_ref[...] = x_ref[...] * dinv


def _tc_prep(deg2, x_pad):
    return pl.pallas_call(
        _tc_prep_body,
        grid=(NBLK,),
        in_specs=[
            pl.BlockSpec((NC, BLK, 1), lambda i: (0, i, 0)),
            pl.BlockSpec((BLK, D), lambda i: (i, 0)),
        ],
        out_specs=[
            pl.BlockSpec((BLK, 1), lambda i: (i, 0)),
            pl.BlockSpec((BLK, D), lambda i: (i, 0)),
        ],
        out_shape=[
            jax.ShapeDtypeStruct((NPAD, 1), jnp.float32),
            jax.ShapeDtypeStruct((NPAD, D), jnp.float32),
        ],
    )(deg2, x_pad)


def _tc_mid_body(x_ref, s_ref, dinv_ref, w0_ref, w1_ref, b_ref, gw_ref,
                 as_ref, ad_ref, xw_ref, asrc_ref, adst_ref, mx_ref, mxs):
    i = pl.program_id(0)
    st = s_ref[0] + s_ref[1]
    tx1 = -(dinv_ref[...] * st)
    h1 = jnp.dot(x_ref[...], w0_ref[...], preferred_element_type=jnp.float32)
    h1 += jnp.dot(tx1, w1_ref[...], preferred_element_type=jnp.float32)
    h1 = jnp.maximum(h1 + b_ref[...], 0.0)
    xw = jnp.dot(h1, gw_ref[...], preferred_element_type=jnp.float32)
    xw_ref[...] = xw
    asrc = jnp.dot(xw, as_ref[...], preferred_element_type=jnp.float32)
    adst = jnp.dot(xw, ad_ref[...], preferred_element_type=jnp.float32)
    asrc_ref[...] = asrc
    adst_ref[...] = adst
    m = jnp.max(asrc)

    @pl.when(i == 0)
    def _():
        mxs[0] = m

    @pl.when(i > 0)
    def _():
        mxs[0] = jnp.maximum(mxs[0], m)

    mx_ref[...] = jnp.full((1, 1), mxs[0], jnp.float32)


def _tc_mid(x_pad, S, dinv, W0, W1, b, gW, att_s, att_d):
    return pl.pallas_call(
        _tc_mid_body,
        grid=(NBLK,),
        in_specs=[
            pl.BlockSpec((BLK, D), lambda i: (i, 0)),
            pl.BlockSpec((NC, BLK, D), lambda i: (0, i, 0)),
            pl.BlockSpec((BLK, 1), lambda i: (i, 0)),
            pl.BlockSpec((D, D), lambda i: (0, 0)),
            pl.BlockSpec((D, D), lambda i: (0, 0)),
            pl.BlockSpec((1, D), lambda i: (0, 0)),
            pl.BlockSpec((D, D), lambda i: (0, 0)),
            pl.BlockSpec((D, 1), lambda i: (0, 0)),
            pl.BlockSpec((D, 1), lambda i: (0, 0)),
        ],
        out_specs=[
            pl.BlockSpec((BLK, D), lambda i: (i, 0)),
            pl.BlockSpec((BLK, 1), lambda i: (i, 0)),
            pl.BlockSpec((BLK, 1), lambda i: (i, 0)),
            pl.BlockSpec((1, 1), lambda i: (0, 0)),
        ],
        out_shape=[
            jax.ShapeDtypeStruct((NPAD, D), jnp.float32),
            jax.ShapeDtypeStruct((NPAD, 1), jnp.float32),
            jax.ShapeDtypeStruct((NPAD, 1), jnp.float32),
            jax.ShapeDtypeStruct((1, 1), jnp.float32),
        ],
        scratch_shapes=[pltpu.SMEM((1,), jnp.float32)],
    )(x_pad, S, dinv, W0, W1, b, gW, att_s, att_d)


def _tc_lstm_body(es_ref, len_ref, wih_ref, whh_ref, bih_ref, bhh_ref,
                  h_ref, xp_ref):
    xp_ref[...] = (jnp.dot(es_ref[...], wih_ref[...],
                           preferred_element_type=jnp.float32)
                   + bih_ref[...] + bhh_ref[...])
    lens = len_ref[...]

    def step(t, hc):
        h, c = hc
        g = xp_ref[pl.ds(t * NG, NG), :] + jnp.dot(
            h, whh_ref[...], preferred_element_type=jnp.float32)
        ii = jax.nn.sigmoid(g[:, :LH])
        ff = jax.nn.sigmoid(g[:, LH:2 * LH])
        gg = jnp.tanh(g[:, 2 * LH:3 * LH])
        oo = jax.nn.sigmoid(g[:, 3 * LH:])
        c_new = ff * c + ii * gg
        h_new = oo * jnp.tanh(c_new)
        mask = t < lens
        return (jnp.where(mask, h_new, h), jnp.where(mask, c_new, c))

    h0 = jnp.zeros((NG, LH), jnp.float32)
    h, _ = lax.fori_loop(0, SEQ, step, (h0, h0))
    h_ref[...] = h


def _tc_lstm(es2, lens, wihT, whhT, bih, bhh):
    return pl.pallas_call(
        _tc_lstm_body,
        out_shape=jax.ShapeDtypeStruct((NG, LH), jnp.float32),
        scratch_shapes=[pltpu.VMEM((SEQ * NG, 4 * LH), jnp.float32)],
    )(es2, lens, wihT, whhT, bih, bhh)


def _tc_post_body(num_ref, den_ref, asrc_ref, adst_ref, mx_ref, xw_ref, b_ref,
                  bt_ref, hl_ref, lw_ref, lb_ref, out_ref, pooled, counts):
    i = pl.program_id(0)
    mx = mx_ref[...]
    asrc = asrc_ref[...]
    adst = adst_ref[...]
    m = mx + adst
    m = jnp.where(m > 0, m, 0.2 * m)
    a = asrc + adst
    a = jnp.where(a > 0, a, 0.2 * a)
    u_self = jnp.exp(a - m)
    den = den_ref[0] + den_ref[1] + u_self
    num = num_ref[0] + num_ref[1] + u_self * xw_ref[...]
    h2 = jnp.maximum(num / jnp.maximum(den, 1e-16) + b_ref[...], 0.0)
    gid = lax.broadcasted_iota(jnp.int32, (NG, BLK), 0)
    onehot = (gid == bt_ref[...]).astype(jnp.float32)
    bp = jnp.dot(onehot, h2, preferred_element_type=jnp.float32)
    bc = jnp.sum(onehot, axis=1, keepdims=True)

    @pl.when(i == 0)
    def _():
        pooled[...] = jnp.zeros_like(pooled)
        counts[...] = jnp.zeros_like(counts)

    pooled[...] += bp
    counts[...] += bc

    @pl.when(i == NBLK - 1)
    def _():
        ge = pooled[...] / jnp.maximum(counts[...], 1.0)
        lw = lw_ref[...]
        out_ref[...] = (jnp.dot(ge, lw[:D], preferred_element_type=jnp.float32)
                        + jnp.dot(hl_ref[...], lw[D:],
                                  preferred_element_type=jnp.float32)
                        + lb_ref[...])


def _tc_post(num, den2, asrc, adst, mx, xw, gb, batch_row, h_lstm, lwT, lb):
    return pl.pallas_call(
        _tc_post_body,
        grid=(NBLK,),
        in_specs=[
            pl.BlockSpec((NC, BLK, D), lambda i: (0, i, 0)),
            pl.BlockSpec((NC, BLK, 1), lambda i: (0, i, 0)),
            pl.BlockSpec((BLK, 1), lambda i: (i, 0)),
            pl.BlockSpec((BLK, 1), lambda i: (i, 0)),
            pl.BlockSpec((1, 1), lambda i: (0, 0)),
            pl.BlockSpec((BLK, D), lambda i: (i, 0)),
            pl.BlockSpec((1, D), lambda i: (0, 0)),
            pl.BlockSpec((1, BLK), lambda i: (0, i)),
            pl.BlockSpec((NG, LH), lambda i: (0, 0)),
            pl.BlockSpec((D + LH, 1), lambda i: (0, 0)),
            pl.BlockSpec((1, 1), lambda i: (0, 0)),
        ],
        out_specs=pl.BlockSpec((NG, 1), lambda i: (0, 0)),
        out_shape=jax.ShapeDtypeStruct((NG, 1), jnp.float32),
        scratch_shapes=[
            pltpu.VMEM((NG, D), jnp.float32),
            pltpu.VMEM((NG, 1), jnp.float32),
        ],
    )(num, den2, asrc, adst, mx, xw, gb, batch_row, h_lstm, lwT, lb)


def kernel(x, edge_index, batch, edge_seq, edge_seq_lengths,
           cheb_W0, cheb_W1, cheb_b, gat_W, gat_att_src, gat_att_dst, gat_b,
           W_ih, W_hh, b_ih, b_hh, lin_W, lin_b):
    x_pad = jnp.pad(x, ((0, NPAD - N), (0, 0)))
    batch_row = jnp.pad(batch.astype(jnp.int32), (0, NPAD - N),
                        constant_values=NG).reshape(1, NPAD)
    erow = edge_index[0].astype(jnp.int32)
    ecol = edge_index[1].astype(jnp.int32)
    zn = jnp.zeros((NPAD,), jnp.float32)
    znd = jnp.zeros((NPAD, D), jnp.float32)

    h_lstm = _tc_lstm(
        edge_seq.transpose(1, 0, 2).reshape(SEQ * NG, 2),
        edge_seq_lengths.astype(jnp.int32).reshape(NG, 1),
        W_ih.T, W_hh.T,
        b_ih.reshape(1, 4 * LH), b_hh.reshape(1, 4 * LH))

    deg2 = _sc_deg(erow, zn)
    dinv, y = _tc_prep(deg2.reshape(NC, NPAD, 1), x_pad)
    S = _sc_cheb(y, erow, ecol, znd)
    xw, asrc, adst, mx = _tc_mid(
        x_pad, S, dinv, cheb_W0, cheb_W1, cheb_b.reshape(1, D), gat_W,
        gat_att_src.reshape(D, 1), gat_att_dst.reshape(D, 1))
    mx16 = jnp.full((16,), mx[0, 0], jnp.float32)
    num, den = _sc_gat(xw, erow, ecol, asrc.reshape(NPAD), adst.reshape(NPAD),
                       mx16, znd, zn)
    out = _tc_post(num, den.reshape(NC, NPAD, 1), asrc, adst, mx, xw,
                   gat_b.reshape(1, D), batch_row, h_lstm,
                   lin_W.T, lin_b.reshape(1, 1))
    return out


# speedup vs baseline: 37.3072x; 1.1856x over previous
import functools

import jax
import jax.numpy as jnp
from jax import lax
from jax.experimental import pallas as pl
from jax.experimental.pallas import tpu as pltpu
from jax.experimental.pallas import tpu_sc as plsc

N = 10000
NPAD = 10240
E = 320000
D = 128
NG = 16
SEQ = 200
LH = 128

NC = 2
NS = 16
EPT = E // (NC * NS)
CHUNK = 80
NCHUNKS = EPT // CHUNK
NQUADS = (NCHUNKS - 1) // 4
NODES_PER_TILE = NPAD // NS
BLK = 1024
NBLK = NPAD // BLK

_SC_MESH = plsc.VectorSubcoreMesh(core_axis_name="c", subcore_axis_name="s")


@functools.partial(
    pl.kernel,
    out_type=jax.ShapeDtypeStruct((NC, NPAD), jnp.float32),
    mesh=_SC_MESH,
    scratch_types=[
        pltpu.VMEM_SHARED((NPAD,), jnp.float32),
        pltpu.VMEM((CHUNK,), jnp.int32),
        pltpu.VMEM((CHUNK,), jnp.int32),
        pltpu.VMEM((CHUNK,), jnp.int32),
        pltpu.VMEM((CHUNK,), jnp.int32),
        pltpu.VMEM((CHUNK,), jnp.float32),
        pltpu.SemaphoreType.DMA,
        pltpu.SemaphoreType.DMA,
        pltpu.SemaphoreType.DMA,
    ],
)
def _sc_deg(erow_hbm, zn_hbm, out_hbm, deg_sh,
            idx0, idx1, idx2, idx3, ones_v, isem, sem0, sem1):
    c = lax.axis_index("c")
    s = lax.axis_index("s")
    nbase = s * NODES_PER_TILE
    ebase = c * (E // NC) + s * EPT
    pltpu.sync_copy(zn_hbm.at[pl.ds(nbase, NODES_PER_TILE)],
                    deg_sh.at[pl.ds(nbase, NODES_PER_TILE)])

    def fill(j, _):
        ones_v[pl.ds(j * 16, 16)] = jnp.ones((16,), jnp.float32)
        return 0

    lax.fori_loop(0, CHUNK // 16, fill, 0)
    plsc.subcore_barrier()
    idxs = (idx0, idx1, idx2, idx3)
    sems = (sem0, sem1)

    def ehbm(k):
        return erow_hbm.at[pl.ds(ebase + k * CHUNK, CHUNK)]

    pltpu.sync_copy(ehbm(0), idx0)

    def quad(qq, _):
        for b4 in range(4):
            k = 4 * qq + b4
            b = b4 % 2
            j = b4
            jn = (b4 + 1) % 4

            @pl.when(k >= 2)
            def _():
                pltpu.make_async_copy(
                    ones_v, deg_sh.at[idxs[(j + 2) % 4]], sems[b]).wait()

            if b4 == 0:
                @pl.when(qq > 0)
                def _():
                    pltpu.make_async_copy(ehbm(k), idxs[j], isem).wait()
            else:
                pltpu.make_async_copy(ehbm(k), idxs[j], isem).wait()
            pltpu.async_copy(ehbm(k + 1), idxs[jn], isem)
            pltpu.async_copy(ones_v, deg_sh.at[idxs[j]], sems[b], add=True)
        return 0

    lax.fori_loop(0, NQUADS, quad, 0)
    pltpu.make_async_copy(ones_v, deg_sh.at[idxs[2]], sem0).wait()
    pltpu.make_async_copy(ones_v, deg_sh.at[idxs[3]], sem1).wait()
    pltpu.make_async_copy(ehbm(NCHUNKS - 1), idx0, isem).wait()
    pltpu.sync_copy(ones_v, deg_sh.at[idx0], add=True)
    plsc.subcore_barrier()
    pltpu.sync_copy(deg_sh.at[pl.ds(nbase, NODES_PER_TILE)],
                    out_hbm.at[c, pl.ds(nbase, NODES_PER_TILE)])


@functools.partial(
    pl.kernel,
    out_type=jax.ShapeDtypeStruct((NC, NPAD, D), jnp.float32),
    mesh=_SC_MESH,
    scratch_types=[
        pltpu.VMEM_SHARED((NPAD, D), jnp.float32),
        pltpu.VMEM((EPT,), jnp.int32),
        pltpu.VMEM((CHUNK,), jnp.int32),
        pltpu.VMEM((CHUNK,), jnp.int32),
        pltpu.VMEM((CHUNK,), jnp.int32),
        pltpu.VMEM((CHUNK,), jnp.int32),
        pltpu.VMEM((CHUNK, D), jnp.float32),
        pltpu.VMEM((CHUNK, D), jnp.float32),
        pltpu.SemaphoreType.DMA,
        pltpu.SemaphoreType.DMA,
        pltpu.SemaphoreType.DMA,
        pltpu.SemaphoreType.DMA,
    ],
)
def _sc_cheb(y_hbm, erow_hbm, ecol_hbm, znd_hbm, out_hbm, acc_sh,
             ridx_all, cidx0, cidx1, cidx2, cidx3, rows0, rows1,
             isem, gsem, ssem0, ssem1):
    c = lax.axis_index("c")
    s = lax.axis_index("s")
    nbase = s * NODES_PER_TILE
    ebase = c * (E // NC) + s * EPT
    pltpu.sync_copy(erow_hbm.at[pl.ds(ebase, EPT)], ridx_all)
    pltpu.sync_copy(znd_hbm.at[pl.ds(nbase, NODES_PER_TILE)],
                    acc_sh.at[pl.ds(nbase, NODES_PER_TILE)])
    plsc.subcore_barrier()
    cidx = (cidx0, cidx1, cidx2, cidx3)
    rows = (rows0, rows1)
    ssem = (ssem0, ssem1)

    def chbm(k):
        return ecol_hbm.at[pl.ds(ebase + k * CHUNK, CHUNK)]

    def rsl(k):
        return ridx_all.at[pl.ds(k * CHUNK, CHUNK)]

    pltpu.sync_copy(chbm(0), cidx0)
    pltpu.async_copy(y_hbm.at[rsl(0)], rows0, gsem)

    def quad(qq, _):
        for b4 in range(4):
            k = 4 * qq + b4
            b = b4 % 2
            nb = 1 - b
            j = b4
            jn = (b4 + 1) % 4

            @pl.when(k >= 1)
            def _():
                pltpu.make_async_copy(
                    rows[nb], acc_sh.at[cidx[(j + 3) % 4]], ssem[nb]).wait()

            pltpu.make_async_copy(y_hbm.at[rsl(k)], rows[b], gsem).wait()
            pltpu.async_copy(y_hbm.at[rsl(k + 1)], rows[nb], gsem)
            if b4 == 0:
                @pl.when(qq > 0)
                def _():
                    pltpu.make_async_copy(chbm(k), cidx[j], isem).wait()
            else:
                pltpu.make_async_copy(chbm(k), cidx[j], isem).wait()
            pltpu.async_copy(chbm(k + 1), cidx[jn], isem)
            pltpu.async_copy(rows[b], acc_sh.at[cidx[j]], ssem[b], add=True)
        return 0

    lax.fori_loop(0, NQUADS, quad, 0)
    pltpu.make_async_copy(
        rows1, acc_sh.at[cidx3], ssem1).wait()
    pltpu.make_async_copy(y_hbm.at[rsl(NCHUNKS - 1)], rows0, gsem).wait()
    pltpu.make_async_copy(chbm(NCHUNKS - 1), cidx0, isem).wait()
    pltpu.sync_copy(rows0, acc_sh.at[cidx0], add=True)
    plsc.subcore_barrier()
    pltpu.sync_copy(acc_sh.at[pl.ds(nbase, NODES_PER_TILE)],
                    out_hbm.at[c, pl.ds(nbase, NODES_PER_TILE)])


@functools.partial(
    pl.kernel,
    out_type=(
        jax.ShapeDtypeStruct((NC, NPAD, D), jnp.float32),
        jax.ShapeDtypeStruct((NC, NPAD), jnp.float32),
    ),
    mesh=_SC_MESH,
    scratch_types=[
        pltpu.VMEM_SHARED((NPAD, D), jnp.float32),
        pltpu.VMEM_SHARED((NPAD,), jnp.float32),
        pltpu.VMEM((EPT,), jnp.int32),
        pltpu.VMEM((CHUNK,), jnp.int32),
        pltpu.VMEM((CHUNK,), jnp.int32),
        pltpu.VMEM((CHUNK,), jnp.int32),
        pltpu.VMEM((CHUNK,), jnp.int32),
        pltpu.VMEM((CHUNK, D), jnp.float32),
        pltpu.VMEM((CHUNK, D), jnp.float32),
        pltpu.VMEM((CHUNK,), jnp.float32),
        pltpu.VMEM((CHUNK,), jnp.float32),
        pltpu.VMEM((CHUNK,), jnp.float32),
        pltpu.VMEM((CHUNK,), jnp.float32),
        pltpu.VMEM((CHUNK,), jnp.float32),
        pltpu.VMEM((CHUNK,), jnp.float32),
        pltpu.VMEM((16,), jnp.float32),
        pltpu.SemaphoreType.DMA,
        pltpu.SemaphoreType.DMA,
        pltpu.SemaphoreType.DMA,
        pltpu.SemaphoreType.DMA,
        pltpu.SemaphoreType.DMA,
        pltpu.SemaphoreType.DMA,
        pltpu.SemaphoreType.DMA,
        pltpu.SemaphoreType.DMA,
    ],
)
def _sc_gat(xw_hbm, erow_hbm, ecol_hbm, asrc_hbm, adst_hbm, mx_hbm, znd_hbm,
            zn_hbm, num_hbm, den_hbm,
            acc_sh, den_sh, ridx_all, cidx0, cidx1, cidx2, cidx3,
            rows0, rows1, av0, av1, dv0, dv1, u0, u1, mx_v,
            isem, asem, dsem, rsem, ssem0, ssem1, usem0, usem1):
    c = lax.axis_index("c")
    s = lax.axis_index("s")
    nbase = s * NODES_PER_TILE
    ebase = c * (E // NC) + s * EPT
    pltpu.sync_copy(erow_hbm.at[pl.ds(ebase, EPT)], ridx_all)
    pltpu.sync_copy(znd_hbm.at[pl.ds(nbase, NODES_PER_TILE)],
                    acc_sh.at[pl.ds(nbase, NODES_PER_TILE)])
    pltpu.sync_copy(zn_hbm.at[pl.ds(nbase, NODES_PER_TILE)],
                    den_sh.at[pl.ds(nbase, NODES_PER_TILE)])
    pltpu.sync_copy(mx_hbm, mx_v)
    plsc.subcore_barrier()
    cidx = (cidx0, cidx1, cidx2, cidx3)
    rows = (rows0, rows1)
    av = (av0, av1)
    dv = (dv0, dv1)
    u = (u0, u1)
    ssem = (ssem0, ssem1)
    usem = (usem0, usem1)

    def chbm(k):
        return ecol_hbm.at[pl.ds(ebase + k * CHUNK, CHUNK)]

    def rsl(k):
        return ridx_all.at[pl.ds(k * CHUNK, CHUNK)]

    pltpu.sync_copy(chbm(0), cidx0)
    pltpu.async_copy(chbm(1), cidx1, isem)
    pltpu.async_copy(asrc_hbm.at[rsl(0)], av0, asem)
    pltpu.async_copy(adst_hbm.at[cidx0], dv0, dsem)
    pltpu.async_copy(xw_hbm.at[rsl(0)], rows0, rsem)

    def vec_u(ub, avb, dvb):
        mx = mx_v[...]

        @plsc.parallel_loop(0, CHUNK // 16, 1, unroll=CHUNK // 16)
        def _(i):
            sl = pl.ds(i * 16, 16)
            a = avb[sl] + dvb[sl]
            alpha = jnp.where(a > 0, a, 0.2 * a)
            m = mx + dvb[sl]
            m = jnp.where(m > 0, m, 0.2 * m)
            ub[sl] = jnp.exp(alpha - m)

    def scale_rows(rb, ub):
        @plsc.parallel_loop(0, CHUNK // 16, 1, unroll=1)
        def _(g):
            uvec = ub[pl.ds(g * 16, 16)]
            for r in range(16):
                uu = uvec[r]
                for i in range(D // 16):
                    sl = pl.ds(i * 16, 16)
                    rb[g * 16 + r, sl] = rb[g * 16 + r, sl] * uu

    def quad(qq, _):
        for b4 in range(4):
            k = 4 * qq + b4
            b = b4 % 2
            nb = 1 - b
            j = b4
            jn = (b4 + 1) % 4

            pltpu.make_async_copy(asrc_hbm.at[rsl(k)], av[b], asem).wait()
            pltpu.make_async_copy(adst_hbm.at[cidx[j]], dv[b], dsem).wait()

            @pl.when(k >= 2)
            def _():
                pltpu.make_async_copy(
                    u[b], den_sh.at[cidx[(j + 2) % 4]], usem[b]).wait()

            vec_u(u[b], av[b], dv[b])
            pltpu.async_copy(u[b], den_sh.at[cidx[j]], usem[b], add=True)

            pltpu.make_async_copy(chbm(k + 1), cidx[jn], isem).wait()

            @pl.when(k + 2 < NCHUNKS)
            def _():
                pltpu.async_copy(chbm(k + 2), cidx[(b4 + 2) % 4], isem)
            pltpu.async_copy(asrc_hbm.at[rsl(k + 1)], av[nb], asem)
            pltpu.async_copy(adst_hbm.at[cidx[jn]], dv[nb], dsem)

            pltpu.make_async_copy(xw_hbm.at[rsl(k)], rows[b], rsem).wait()

            @pl.when(k >= 1)
            def _():
                pltpu.make_async_copy(
                    rows[nb], acc_sh.at[cidx[(j + 3) % 4]], ssem[nb]).wait()

            pltpu.async_copy(xw_hbm.at[rsl(k + 1)], rows[nb], rsem)
            scale_rows(rows[b], u[b])
            pltpu.async_copy(rows[b], acc_sh.at[cidx[j]], ssem[b], add=True)
        return 0

    lax.fori_loop(0, NQUADS, quad, 0)
    k = NCHUNKS - 1
    pltpu.make_async_copy(asrc_hbm.at[rsl(k)], av0, asem).wait()
    pltpu.make_async_copy(adst_hbm.at[cidx0], dv0, dsem).wait()
    pltpu.make_async_copy(u0, den_sh.at[cidx2], usem0).wait()
    vec_u(u0, av0, dv0)
    pltpu.sync_copy(u0, den_sh.at[cidx0], add=True)
    pltpu.make_async_copy(xw_hbm.at[rsl(k)], rows0, rsem).wait()
    pltpu.make_async_copy(rows1, acc_sh.at[cidx3], ssem1).wait()
    scale_rows(rows0, u0)
    pltpu.sync_copy(rows0, acc_sh.at[cidx0], add=True)
    pltpu.make_async_copy(u1, den_sh.at[cidx3], usem1).wait()
    plsc.subcore_barrier()
    pltpu.sync_copy(acc_sh.at[pl.ds(nbase, NODES_PER_TILE)],
                    num_hbm.at[c, pl.ds(nbase, NODES_PER_TILE)])
    pltpu.sync_copy(den_sh.at[pl.ds(nbase, NODES_PER_TILE)],
                    den_hbm.at[c, pl.ds(nbase, NODES_PER_TILE)])


def _tc_prep_body(deg_ref, x_ref, dinv_ref, y_ref):
    d = deg_ref[0] + deg_ref[1]
    dinv = jnp.where(d > 0, 1.0 / jnp.sqrt(jnp.maximum(d, 1e-12)), 0.0)
    dinv_ref[...] = dinv
    y_ref[...] = x_ref[...] * dinv


def _tc_prep(deg2, x_pad):
    return pl.pallas_call(
        _tc_prep_body,
        grid=(NBLK,),
        in_specs=[
            pl.BlockSpec((NC, BLK, 1), lambda i: (0, i, 0)),
            pl.BlockSpec((BLK, D), lambda i: (i, 0)),
        ],
        out_specs=[
            pl.BlockSpec((BLK, 1), lambda i: (i, 0)),
            pl.BlockSpec((BLK, D), lambda i: (i, 0)),
        ],
        out_shape=[
            jax.ShapeDtypeStruct((NPAD, 1), jnp.float32),
            jax.ShapeDtypeStruct((NPAD, D), jnp.float32),
        ],
    )(deg2, x_pad)


def _tc_mid_body(x_ref, s_ref, dinv_ref, w0_ref, w1_ref, b_ref, gw_ref,
                 as_ref, ad_ref, xw_ref, asrc_ref, adst_ref, mx_ref, mxs):
    i = pl.program_id(0)
    st = s_ref[0] + s_ref[1]
    tx1 = -(dinv_ref[...] * st)
    h1 = jnp.dot(x_ref[...], w0_ref[...], preferred_element_type=jnp.float32)
    h1 += jnp.dot(tx1, w1_ref[...], preferred_element_type=jnp.float32)
    h1 = jnp.maximum(h1 + b_ref[...], 0.0)
    xw = jnp.dot(h1, gw_ref[...], preferred_element_type=jnp.float32)
    xw_ref[...] = xw
    asrc = jnp.dot(xw, as_ref[...], preferred_element_type=jnp.float32)
    adst = jnp.dot(xw, ad_ref[...], preferred_element_type=jnp.float32)
    asrc_ref[...] = asrc
    adst_ref[...] = adst
    m = jnp.max(asrc)

    @pl.when(i == 0)
    def _():
        mxs[0] = m

    @pl.when(i > 0)
    def _():
        mxs[0] = jnp.maximum(mxs[0], m)

    mx_ref[...] = jnp.full((1, 1), mxs[0], jnp.float32)


def _tc_mid(x_pad, S, dinv, W0, W1, b, gW, att_s, att_d):
    return pl.pallas_call(
        _tc_mid_body,
        grid=(NBLK,),
        in_specs=[
            pl.BlockSpec((BLK, D), lambda i: (i, 0)),
            pl.BlockSpec((NC, BLK, D), lambda i: (0, i, 0)),
            pl.BlockSpec((BLK, 1), lambda i: (i, 0)),
            pl.BlockSpec((D, D), lambda i: (0, 0)),
            pl.BlockSpec((D, D), lambda i: (0, 0)),
            pl.BlockSpec((1, D), lambda i: (0, 0)),
            pl.BlockSpec((D, D), lambda i: (0, 0)),
            pl.BlockSpec((D, 1), lambda i: (0, 0)),
            pl.BlockSpec((D, 1), lambda i: (0, 0)),
        ],
        out_specs=[
            pl.BlockSpec((BLK, D), lambda i: (i, 0)),
            pl.BlockSpec((BLK, 1), lambda i: (i, 0)),
            pl.BlockSpec((BLK, 1), lambda i: (i, 0)),
            pl.BlockSpec((1, 1), lambda i: (0, 0)),
        ],
        out_shape=[
            jax.ShapeDtypeStruct((NPAD, D), jnp.float32),
            jax.ShapeDtypeStruct((NPAD, 1), jnp.float32),
            jax.ShapeDtypeStruct((NPAD, 1), jnp.float32),
            jax.ShapeDtypeStruct((1, 1), jnp.float32),
        ],
        scratch_shapes=[pltpu.SMEM((1,), jnp.float32)],
    )(x_pad, S, dinv, W0, W1, b, gW, att_s, att_d)


def _tc_lstm_body(es_ref, len_ref, wih_ref, whh_ref, bih_ref, bhh_ref,
                  h_ref, xp_ref):
    xp_ref[...] = (jnp.dot(es_ref[...], wih_ref[...],
                           preferred_element_type=jnp.float32)
                   + bih_ref[...] + bhh_ref[...])
    lens = len_ref[...]

    def step(t, hc):
        h, c = hc
        g = xp_ref[pl.ds(t * NG, NG), :] + jnp.dot(
            h, whh_ref[...], preferred_element_type=jnp.float32)
        ii = jax.nn.sigmoid(g[:, :LH])
        ff = jax.nn.sigmoid(g[:, LH:2 * LH])
        gg = jnp.tanh(g[:, 2 * LH:3 * LH])
        oo = jax.nn.sigmoid(g[:, 3 * LH:])
        c_new = ff * c + ii * gg
        h_new = oo * jnp.tanh(c_new)
        mask = t < lens
        return (jnp.where(mask, h_new, h), jnp.where(mask, c_new, c))

    h0 = jnp.zeros((NG, LH), jnp.float32)
    h, _ = lax.fori_loop(0, SEQ, step, (h0, h0))
    h_ref[...] = h


def _tc_lstm(es2, lens, wihT, whhT, bih, bhh):
    return pl.pallas_call(
        _tc_lstm_body,
        out_shape=jax.ShapeDtypeStruct((NG, LH), jnp.float32),
        scratch_shapes=[pltpu.VMEM((SEQ * NG, 4 * LH), jnp.float32)],
    )(es2, lens, wihT, whhT, bih, bhh)


def _tc_post_body(num_ref, den_ref, asrc_ref, adst_ref, mx_ref, xw_ref, b_ref,
                  bt_ref, hl_ref, lw_ref, lb_ref, out_ref, pooled, counts):
    i = pl.program_id(0)
    mx = mx_ref[...]
    asrc = asrc_ref[...]
    adst = adst_ref[...]
    m = mx + adst
    m = jnp.where(m > 0, m, 0.2 * m)
    a = asrc + adst
    a = jnp.where(a > 0, a, 0.2 * a)
    u_self = jnp.exp(a - m)
    den = den_ref[0] + den_ref[1] + u_self
    num = num_ref[0] + num_ref[1] + u_self * xw_ref[...]
    h2 = jnp.maximum(num / jnp.maximum(den, 1e-16) + b_ref[...], 0.0)
    gid = lax.broadcasted_iota(jnp.int32, (NG, BLK), 0)
    onehot = (gid == bt_ref[...]).astype(jnp.float32)
    bp = jnp.dot(onehot, h2, preferred_element_type=jnp.float32)
    bc = jnp.sum(onehot, axis=1, keepdims=True)

    @pl.when(i == 0)
    def _():
        pooled[...] = jnp.zeros_like(pooled)
        counts[...] = jnp.zeros_like(counts)

    pooled[...] += bp
    counts[...] += bc

    @pl.when(i == NBLK - 1)
    def _():
        ge = pooled[...] / jnp.maximum(counts[...], 1.0)
        lw = lw_ref[...]
        out_ref[...] = (jnp.dot(ge, lw[:D], preferred_element_type=jnp.float32)
                        + jnp.dot(hl_ref[...], lw[D:],
                                  preferred_element_type=jnp.float32)
                        + lb_ref[...])


def _tc_post(num, den2, asrc, adst, mx, xw, gb, batch_row, h_lstm, lwT, lb):
    return pl.pallas_call(
        _tc_post_body,
        grid=(NBLK,),
        in_specs=[
            pl.BlockSpec((NC, BLK, D), lambda i: (0, i, 0)),
            pl.BlockSpec((NC, BLK, 1), lambda i: (0, i, 0)),
            pl.BlockSpec((BLK, 1), lambda i: (i, 0)),
            pl.BlockSpec((BLK, 1), lambda i: (i, 0)),
            pl.BlockSpec((1, 1), lambda i: (0, 0)),
            pl.BlockSpec((BLK, D), lambda i: (i, 0)),
            pl.BlockSpec((1, D), lambda i: (0, 0)),
            pl.BlockSpec((1, BLK), lambda i: (0, i)),
            pl.BlockSpec((NG, LH), lambda i: (0, 0)),
            pl.BlockSpec((D + LH, 1), lambda i: (0, 0)),
            pl.BlockSpec((1, 1), lambda i: (0, 0)),
        ],
        out_specs=pl.BlockSpec((NG, 1), lambda i: (0, 0)),
        out_shape=jax.ShapeDtypeStruct((NG, 1), jnp.float32),
        scratch_shapes=[
            pltpu.VMEM((NG, D), jnp.float32),
            pltpu.VMEM((NG, 1), jnp.float32),
        ],
    )(num, den2, asrc, adst, mx, xw, gb, batch_row, h_lstm, lwT, lb)


def kernel(x, edge_index, batch, edge_seq, edge_seq_lengths,
           cheb_W0, cheb_W1, cheb_b, gat_W, gat_att_src, gat_att_dst, gat_b,
           W_ih, W_hh, b_ih, b_hh, lin_W, lin_b):
    x_pad = jnp.pad(x, ((0, NPAD - N), (0, 0)))
    batch_row = jnp.pad(batch.astype(jnp.int32), (0, NPAD - N),
                        constant_values=NG).reshape(1, NPAD)
    erow = edge_index[0].astype(jnp.int32)
    ecol = edge_index[1].astype(jnp.int32)
    zn = jnp.zeros((NPAD,), jnp.float32)
    znd = jnp.zeros((NPAD, D), jnp.float32)

    h_lstm = _tc_lstm(
        edge_seq.transpose(1, 0, 2).reshape(SEQ * NG, 2),
        edge_seq_lengths.astype(jnp.int32).reshape(NG, 1),
        W_ih.T, W_hh.T,
        b_ih.reshape(1, 4 * LH), b_hh.reshape(1, 4 * LH))

    deg2 = _sc_deg(erow, zn)
    dinv, y = _tc_prep(deg2.reshape(NC, NPAD, 1), x_pad)
    S = _sc_cheb(y, erow, ecol, znd)
    xw, asrc, adst, mx = _tc_mid(
        x_pad, S, dinv, cheb_W0, cheb_W1, cheb_b.reshape(1, D), gat_W,
        gat_att_src.reshape(D, 1), gat_att_dst.reshape(D, 1))
    mx16 = jnp.full((16,), mx[0, 0], jnp.float32)
    num, den = _sc_gat(xw, erow, ecol, asrc.reshape(NPAD), adst.reshape(NPAD),
                       mx16, znd, zn)
    out = _tc_post(num, den.reshape(NC, NPAD, 1), asrc, adst, mx, xw,
                   gat_b.reshape(1, D), batch_row, h_lstm,
                   lin_W.T, lin_b.reshape(1, 1))
    return out
